# Initial kernel scaffold; baseline (speedup 1.0000x reference)
import functools

import jax
import jax.numpy as jnp
from jax import lax
from jax.experimental import pallas as pl
from jax.experimental.pallas import tpu as pltpu
from jax.experimental.pallas import tpu_sc as plsc

N_NODES = 10000
CH = 128
N_PAD = 10240
NW = 32
CHUNK = 128
ROWS_PER_TILE = N_PAD // 16
DRAIN_CH = ROWS_PER_TILE // CHUNK


def _mesh():
    return plsc.VectorSubcoreMesh(core_axis_name="c", subcore_axis_name="s")


def _deg_count_kernel(nch):
    @functools.partial(
        pl.kernel,
        out_type=jax.ShapeDtypeStruct((2, N_PAD, 16), jnp.float32),
        mesh=_mesh(),
        scratch_types=[
            pltpu.VMEM((nch, CHUNK), jnp.int32),
            pltpu.VMEM((CHUNK, 16), jnp.float32),
            pltpu.VMEM((ROWS_PER_TILE, 16), jnp.float32),
            pltpu.VMEM_SHARED((N_PAD, 16), jnp.float32),
        ],
    )
    def k(row_hbm, ones_hbm, zero_hbm, cnt_hbm, idx_v, ones_v, zv, acc_sh):
        c = lax.axis_index("c")
        s = lax.axis_index("s")
        wid = c * 16 + s
        base = s * ROWS_PER_TILE
        pltpu.sync_copy(row_hbm.at[wid], idx_v)
        pltpu.sync_copy(ones_hbm, ones_v)
        pltpu.sync_copy(zero_hbm, zv)
        pltpu.sync_copy(zv, acc_sh.at[pl.ds(base, ROWS_PER_TILE)])
        plsc.subcore_barrier()

        def body(j, _):
            pltpu.sync_copy(ones_v, acc_sh.at[idx_v.at[j]], add=True)
            return 0

        lax.fori_loop(0, nch, body, 0)
        plsc.subcore_barrier()
        pltpu.sync_copy(acc_sh.at[pl.ds(base, ROWS_PER_TILE)], zv)
        pltpu.sync_copy(zv, cnt_hbm.at[c, pl.ds(base, ROWS_PER_TILE)])

    return k


def _scatter_kernel(nch):
    @functools.partial(
        pl.kernel,
        out_type=jax.ShapeDtypeStruct((2, N_PAD, CH), jnp.float32),
        mesh=_mesh(),
        scratch_types=[
            pltpu.VMEM((nch, CHUNK), jnp.int32),
            pltpu.VMEM((nch, CHUNK), jnp.int32),
            pltpu.VMEM((CHUNK, CH), jnp.float32),
            pltpu.VMEM_SHARED((N_PAD, CH), jnp.float32),
            pltpu.SemaphoreType.DMA,
        ],
    )
    def k(row_hbm, col_hbm, y_hbm, zero_hbm, acc_hbm,
          row_v, col_v, buf, acc_sh, sem):
        c = lax.axis_index("c")
        s = lax.axis_index("s")
        wid = c * 16 + s
        base = s * ROWS_PER_TILE
        pltpu.sync_copy(row_hbm.at[wid], row_v)
        pltpu.sync_copy(col_hbm.at[wid], col_v)
        pltpu.sync_copy(zero_hbm, buf)
        for t in range(DRAIN_CH):
            pltpu.sync_copy(buf, acc_sh.at[pl.ds(base + t * CHUNK, CHUNK)])
        plsc.subcore_barrier()

        def body(j, _):
            pltpu.async_copy(y_hbm.at[col_v.at[j]], buf, sem).wait()
            pltpu.sync_copy(buf, acc_sh.at[row_v.at[j]], add=True)
            return 0

        lax.fori_loop(0, nch, body, 0)
        plsc.subcore_barrier()
        for t in range(DRAIN_CH):
            pltpu.sync_copy(acc_sh.at[pl.ds(base + t * CHUNK, CHUNK)], buf)
            pltpu.sync_copy(buf, acc_hbm.at[c, pl.ds(base + t * CHUNK, CHUNK)])

    return k


def _matmul_body(x_ref, w_ref, cnt_ref, y_ref):
    cnt = cnt_ref[0][:, :1] + cnt_ref[1][:, :1] + 1.0
    dinv = lax.rsqrt(cnt)
    y_ref[...] = jnp.dot(x_ref[...], w_ref[...],
                         preferred_element_type=jnp.float32) * dinv


def _combine_body(acc_ref, y_ref, cnt_ref, out_ref):
    cnt = cnt_ref[0][:, :1] + cnt_ref[1][:, :1] + 1.0
    dinv = lax.rsqrt(cnt)
    total = (acc_ref[0] + acc_ref[1] + y_ref[...]) * dinv
    out_ref[...] = jnp.maximum(total, 0.0)


def _gcn_single(x, row, col, W):
    E = row.shape[0]
    nch = -(-E // (NW * CHUNK))
    e_pad = NW * CHUNK * nch
    npe = e_pad - E
    i32 = jnp.int32
    pad_r = N_NODES + (jnp.arange(npe, dtype=i32) % (N_PAD - N_NODES))
    pad_c = jnp.arange(npe, dtype=i32) % CHUNK
    row3 = jnp.concatenate([row, pad_r]).reshape(NW, nch, CHUNK)
    col3 = jnp.concatenate([col, pad_c]).reshape(NW, nch, CHUNK)
    xp = jnp.pad(x, ((0, N_PAD - x.shape[0]), (0, 0)))

    ones16 = jnp.ones((CHUNK, 16), jnp.float32)
    zeros16 = jnp.zeros((ROWS_PER_TILE, 16), jnp.float32)
    cnt = _deg_count_kernel(nch)(row3, ones16, zeros16)

    rb = 1024
    grid = (N_PAD // rb,)
    y = pl.pallas_call(
        _matmul_body,
        grid=grid,
        in_specs=[
            pl.BlockSpec((rb, CH), lambda i: (i, 0)),
            pl.BlockSpec((CH, CH), lambda i: (0, 0)),
            pl.BlockSpec((2, rb, 16), lambda i: (0, i, 0)),
        ],
        out_specs=pl.BlockSpec((rb, CH), lambda i: (i, 0)),
        out_shape=jax.ShapeDtypeStruct((N_PAD, CH), jnp.float32),
    )(xp, W, cnt)

    zeros128 = jnp.zeros((CHUNK, CH), jnp.float32)
    acc = _scatter_kernel(nch)(row3, col3, y, zeros128)

    out = pl.pallas_call(
        _combine_body,
        grid=grid,
        in_specs=[
            pl.BlockSpec((2, rb, CH), lambda i: (0, i, 0)),
            pl.BlockSpec((rb, CH), lambda i: (i, 0)),
            pl.BlockSpec((2, rb, 16), lambda i: (0, i, 0)),
        ],
        out_specs=pl.BlockSpec((rb, CH), lambda i: (i, 0)),
        out_shape=jax.ShapeDtypeStruct((N_PAD, CH), jnp.float32),
    )(acc, y, cnt)
    return out[:N_NODES]


def kernel(x, edge_index, W):
    B = x.shape[0]
    outs = []
    for i in range(B):
        ei = edge_index[i].astype(jnp.int32)
        outs.append(_gcn_single(x[i], ei[0], ei[1], W))
    return jnp.stack(outs)


# speedup vs baseline: 25.0664x; 25.0664x over previous
import functools

import jax
import jax.numpy as jnp
from jax import lax
from jax.experimental import pallas as pl
from jax.experimental.pallas import tpu as pltpu
from jax.experimental.pallas import tpu_sc as plsc

N_NODES = 10000
CH = 128
N_PAD = 10240
NW = 32
CHUNK = 128
ROWS_PER_TILE = N_PAD // 16
DRAIN_CH = ROWS_PER_TILE // CHUNK


def _mesh():
    return plsc.VectorSubcoreMesh(core_axis_name="c", subcore_axis_name="s",
                                  num_cores=2, num_subcores=16)


def _deg_count_kernel(nch):
    @functools.partial(
        pl.kernel,
        out_type=jax.ShapeDtypeStruct((2, N_PAD, CH), jnp.float32),
        mesh=_mesh(),
        scratch_types=[
            pltpu.VMEM((nch, CHUNK), jnp.int32),
            pltpu.VMEM((DRAIN_CH, CHUNK), jnp.int32),
            pltpu.VMEM((CHUNK, CH), jnp.float32),
            pltpu.VMEM_SHARED((N_PAD, CH), jnp.float32),
            pltpu.SemaphoreType.DMA,
        ],
    )
    def k(row_hbm, zeros_hbm, ones_hbm, lin_hbm, cnt_hbm,
          idx_v, lin_v, buf, acc_sh, sem):
        c = lax.axis_index("c")
        s = lax.axis_index("s")
        wid = c * 16 + s
        base = s * ROWS_PER_TILE
        pltpu.sync_copy(lin_hbm.at[s], lin_v)
        pltpu.sync_copy(row_hbm.at[wid], idx_v)
        pltpu.sync_copy(zeros_hbm, buf)
        for t in range(DRAIN_CH):
            pltpu.sync_copy(buf, acc_sh.at[lin_v.at[t]])
        pltpu.sync_copy(ones_hbm, buf)
        plsc.subcore_barrier()
        for j in range(nch):
            pltpu.sync_copy(buf, acc_sh.at[idx_v.at[j]], add=True)
        plsc.subcore_barrier()
        for t in range(DRAIN_CH):
            pltpu.async_copy(acc_sh.at[lin_v.at[t]], buf, sem).wait()
            pltpu.sync_copy(
                buf, cnt_hbm.at[c, pl.ds(base + t * CHUNK, CHUNK)])

    return k


def _scatter_kernel(nch):
    @functools.partial(
        pl.kernel,
        out_type=jax.ShapeDtypeStruct((2, N_PAD, CH), jnp.float32),
        mesh=_mesh(),
        scratch_types=[
            pltpu.VMEM((nch, CHUNK), jnp.int32),
            pltpu.VMEM((nch, CHUNK), jnp.int32),
            pltpu.VMEM((DRAIN_CH, CHUNK), jnp.int32),
            pltpu.VMEM((CHUNK, CH), jnp.float32),
            pltpu.VMEM_SHARED((N_PAD, CH), jnp.float32),
            pltpu.SemaphoreType.DMA,
        ],
    )
    def k(row_hbm, col_hbm, y_hbm, zeros_hbm, lin_hbm, acc_hbm,
          row_v, col_v, lin_v, buf, acc_sh, sem):
        c = lax.axis_index("c")
        s = lax.axis_index("s")
        wid = c * 16 + s
        base = s * ROWS_PER_TILE
        pltpu.sync_copy(lin_hbm.at[s], lin_v)
        pltpu.sync_copy(row_hbm.at[wid], row_v)
        pltpu.sync_copy(col_hbm.at[wid], col_v)
        pltpu.sync_copy(zeros_hbm, buf)
        for t in range(DRAIN_CH):
            pltpu.sync_copy(buf, acc_sh.at[lin_v.at[t]])
        plsc.subcore_barrier()
        for j in range(nch):
            pltpu.async_copy(y_hbm.at[col_v.at[j]], buf, sem).wait()
            pltpu.sync_copy(buf, acc_sh.at[row_v.at[j]], add=True)
        plsc.subcore_barrier()
        for t in range(DRAIN_CH):
            pltpu.async_copy(acc_sh.at[lin_v.at[t]], buf, sem).wait()
            pltpu.sync_copy(
                buf, acc_hbm.at[c, pl.ds(base + t * CHUNK, CHUNK)])

    return k


def _matmul_body(x_ref, w_ref, cnt_ref, y_ref):
    cnt = cnt_ref[0][:, :1] + cnt_ref[1][:, :1] + 1.0
    dinv = lax.rsqrt(cnt)
    y_ref[...] = jnp.dot(x_ref[...], w_ref[...],
                         preferred_element_type=jnp.float32) * dinv


def _combine_body(acc_ref, y_ref, cnt_ref, out_ref):
    cnt = cnt_ref[0][:, :1] + cnt_ref[1][:, :1] + 1.0
    dinv = lax.rsqrt(cnt)
    total = (acc_ref[0] + acc_ref[1] + y_ref[...]) * dinv
    out_ref[...] = jnp.maximum(total, 0.0)


def _gcn_single(x, row, col, W):
    E = row.shape[0]
    nch = -(-E // (NW * CHUNK))
    e_pad = NW * CHUNK * nch
    npe = e_pad - E
    i32 = jnp.int32
    pad_r = N_NODES + (jnp.arange(npe, dtype=i32) % (N_PAD - N_NODES))
    pad_c = jnp.arange(npe, dtype=i32) % CHUNK
    row3 = jnp.concatenate([row, pad_r]).reshape(NW, nch, CHUNK)
    col3 = jnp.concatenate([col, pad_c]).reshape(NW, nch, CHUNK)
    xp = jnp.pad(x, ((0, N_PAD - x.shape[0]), (0, 0)))
    lin = jnp.arange(N_PAD, dtype=i32).reshape(16, DRAIN_CH, CHUNK)
    zeros = jnp.zeros((CHUNK, CH), jnp.float32)
    ones = jnp.ones((CHUNK, CH), jnp.float32)

    cnt = _deg_count_kernel(nch)(row3, zeros, ones, lin)

    rb = 1024
    grid = (N_PAD // rb,)
    y = pl.pallas_call(
        _matmul_body,
        grid=grid,
        in_specs=[
            pl.BlockSpec((rb, CH), lambda i: (i, 0)),
            pl.BlockSpec((CH, CH), lambda i: (0, 0)),
            pl.BlockSpec((2, rb, CH), lambda i: (0, i, 0)),
        ],
        out_specs=pl.BlockSpec((rb, CH), lambda i: (i, 0)),
        out_shape=jax.ShapeDtypeStruct((N_PAD, CH), jnp.float32),
    )(xp, W, cnt)

    acc = _scatter_kernel(nch)(row3, col3, y, zeros, lin)

    out = pl.pallas_call(
        _combine_body,
        grid=grid,
        in_specs=[
            pl.BlockSpec((2, rb, CH), lambda i: (0, i, 0)),
            pl.BlockSpec((rb, CH), lambda i: (i, 0)),
            pl.BlockSpec((2, rb, CH), lambda i: (0, i, 0)),
        ],
        out_specs=pl.BlockSpec((rb, CH), lambda i: (i, 0)),
        out_shape=jax.ShapeDtypeStruct((N_PAD, CH), jnp.float32),
    )(acc, y, cnt)
    return out[:N_NODES]


def kernel(x, edge_index, W):
    B = x.shape[0]
    outs = []
    for i in range(B):
        ei = edge_index[i].astype(jnp.int32)
        outs.append(_gcn_single(x[i], ei[0], ei[1], W))
    return jnp.stack(outs)


# speedup vs baseline: 31.6306x; 1.2619x over previous
import functools

import jax
import jax.numpy as jnp
from jax import lax
from jax.experimental import pallas as pl
from jax.experimental.pallas import tpu as pltpu
from jax.experimental.pallas import tpu_sc as plsc

N_NODES = 10000
CH = 128
N_PAD = 10240
NW = 32
CHUNK = 128
ROWS_PER_TILE = N_PAD // 16
DRAIN_CH = ROWS_PER_TILE // CHUNK


def _mesh():
    return plsc.VectorSubcoreMesh(core_axis_name="c", subcore_axis_name="s",
                                  num_cores=2, num_subcores=16)


def _deg_count_kernel(nch):
    @functools.partial(
        pl.kernel,
        out_type=jax.ShapeDtypeStruct((2, N_PAD, CH), jnp.float32),
        mesh=_mesh(),
        scratch_types=[
            pltpu.VMEM((nch, CHUNK), jnp.int32),
            pltpu.VMEM((DRAIN_CH, CHUNK), jnp.int32),
            pltpu.VMEM((CHUNK, CH), jnp.float32),
            pltpu.VMEM_SHARED((N_PAD, CH), jnp.float32),
            pltpu.SemaphoreType.DMA,
        ],
    )
    def k(row_hbm, zeros_hbm, ones_hbm, lin_hbm, cnt_hbm,
          idx_v, lin_v, buf, acc_sh, sem):
        c = lax.axis_index("c")
        s = lax.axis_index("s")
        wid = c * 16 + s
        base = s * ROWS_PER_TILE
        pltpu.sync_copy(lin_hbm.at[s], lin_v)
        pltpu.sync_copy(row_hbm.at[wid], idx_v)
        pltpu.sync_copy(zeros_hbm, buf)
        for t in range(DRAIN_CH):
            pltpu.sync_copy(buf, acc_sh.at[lin_v.at[t]])
        pltpu.sync_copy(ones_hbm, buf)
        plsc.subcore_barrier()
        descs = [None] * nch
        for j in range(nch):
            if j >= 8:
                descs[j - 8].wait()
            descs[j] = pltpu.async_copy(
                buf, acc_sh.at[idx_v.at[j]], sem, add=True)
        for j in range(max(0, nch - 8), nch):
            descs[j].wait()
        plsc.subcore_barrier()
        for t in range(DRAIN_CH):
            pltpu.async_copy(acc_sh.at[lin_v.at[t]], buf, sem).wait()
            pltpu.sync_copy(
                buf, cnt_hbm.at[c, pl.ds(base + t * CHUNK, CHUNK)])

    return k


def _scatter_kernel(nch):
    assert nch % 16 == 0
    half = nch // 2

    @functools.partial(
        pl.kernel,
        out_type=jax.ShapeDtypeStruct((2, N_PAD, CH), jnp.float32),
        mesh=_mesh(),
        scratch_types=[
            pltpu.VMEM((half, CHUNK), jnp.int32),
            pltpu.VMEM((nch, CHUNK), jnp.int32),
            pltpu.VMEM((DRAIN_CH, CHUNK), jnp.int32),
            pltpu.VMEM((CHUNK, CH), jnp.float32),
            pltpu.VMEM((CHUNK, CH), jnp.float32),
            pltpu.VMEM_SHARED((N_PAD, CH), jnp.float32),
            pltpu.SemaphoreType.DMA,
            pltpu.SemaphoreType.DMA,
            pltpu.SemaphoreType.DMA,
            pltpu.SemaphoreType.DMA,
        ],
    )
    def k(row_hbm, col_hbm, y_hbm, zeros_hbm, lin_hbm, acc_hbm,
          row_v, col_v, lin_v, buf0, buf1, acc_sh, sg0, sg1, ss0, ss1):
        c = lax.axis_index("c")
        s = lax.axis_index("s")
        wid = c * 16 + s
        base = s * ROWS_PER_TILE
        bufs = (buf0, buf1)
        gsems = (sg0, sg1)
        ssems = (ss0, ss1)
        pltpu.sync_copy(lin_hbm.at[s], lin_v)
        pltpu.sync_copy(col_hbm.at[wid], col_v)
        pltpu.sync_copy(row_hbm.at[wid, pl.ds(0, half)], row_v)
        pltpu.sync_copy(zeros_hbm, buf0)
        for t in range(DRAIN_CH):
            pltpu.sync_copy(buf0, acc_sh.at[lin_v.at[t]])
        plsc.subcore_barrier()
        gat = [None] * nch
        sca = [None] * nch
        gat[0] = pltpu.async_copy(y_hbm.at[col_v.at[0]], bufs[0], gsems[0])
        for j in range(nch):
            b = j % 2
            if j >= 1 and j + 1 < nch:
                sca[j - 1].wait()
            if j == half:
                pltpu.sync_copy(row_hbm.at[wid, pl.ds(half, half)], row_v)
            if j + 1 < nch:
                nb = (j + 1) % 2
                gat[j + 1] = pltpu.async_copy(
                    y_hbm.at[col_v.at[j + 1]], bufs[nb], gsems[nb])
            gat[j].wait()
            ridx = row_v.at[j] if j < half else row_v.at[j - half]
            sca[j] = pltpu.async_copy(
                bufs[b], acc_sh.at[ridx], ssems[b], add=True)
        sca[nch - 2].wait()
        sca[nch - 1].wait()
        plsc.subcore_barrier()
        for t in range(DRAIN_CH):
            pltpu.async_copy(acc_sh.at[lin_v.at[t]], buf0, sg0).wait()
            pltpu.sync_copy(
                buf0, acc_hbm.at[c, pl.ds(base + t * CHUNK, CHUNK)])

    return k


def _matmul_body(x_ref, w_ref, cnt_ref, y_ref):
    cnt = cnt_ref[0][:, :1] + cnt_ref[1][:, :1] + 1.0
    dinv = lax.rsqrt(cnt)
    y_ref[...] = jnp.dot(x_ref[...], w_ref[...],
                         preferred_element_type=jnp.float32) * dinv


def _combine_body(acc_ref, y_ref, cnt_ref, out_ref):
    cnt = cnt_ref[0][:, :1] + cnt_ref[1][:, :1] + 1.0
    dinv = lax.rsqrt(cnt)
    total = (acc_ref[0] + acc_ref[1] + y_ref[...]) * dinv
    out_ref[...] = jnp.maximum(total, 0.0)


def _gcn_single(x, row, col, W):
    E = row.shape[0]
    nch = -(-E // (NW * CHUNK))
    nch = -(-nch // 16) * 16
    e_pad = NW * CHUNK * nch
    npe = e_pad - E
    i32 = jnp.int32
    pad_r = N_NODES + (jnp.arange(npe, dtype=i32) % (N_PAD - N_NODES))
    pad_c = jnp.arange(npe, dtype=i32) % CHUNK
    row3 = jnp.concatenate([row, pad_r]).reshape(NW, nch, CHUNK)
    col3 = jnp.concatenate([col, pad_c]).reshape(NW, nch, CHUNK)
    xp = jnp.pad(x, ((0, N_PAD - x.shape[0]), (0, 0)))
    lin = jnp.arange(N_PAD, dtype=i32).reshape(16, DRAIN_CH, CHUNK)
    zeros = jnp.zeros((CHUNK, CH), jnp.float32)
    ones = jnp.ones((CHUNK, CH), jnp.float32)

    cnt = _deg_count_kernel(nch)(row3, zeros, ones, lin)

    rb = 1024
    grid = (N_PAD // rb,)
    y = pl.pallas_call(
        _matmul_body,
        grid=grid,
        in_specs=[
            pl.BlockSpec((rb, CH), lambda i: (i, 0)),
            pl.BlockSpec((CH, CH), lambda i: (0, 0)),
            pl.BlockSpec((2, rb, CH), lambda i: (0, i, 0)),
        ],
        out_specs=pl.BlockSpec((rb, CH), lambda i: (i, 0)),
        out_shape=jax.ShapeDtypeStruct((N_PAD, CH), jnp.float32),
    )(xp, W, cnt)

    acc = _scatter_kernel(nch)(row3, col3, y, zeros, lin)

    out = pl.pallas_call(
        _combine_body,
        grid=grid,
        in_specs=[
            pl.BlockSpec((2, rb, CH), lambda i: (0, i, 0)),
            pl.BlockSpec((rb, CH), lambda i: (i, 0)),
            pl.BlockSpec((2, rb, CH), lambda i: (0, i, 0)),
        ],
        out_specs=pl.BlockSpec((rb, CH), lambda i: (i, 0)),
        out_shape=jax.ShapeDtypeStruct((N_PAD, CH), jnp.float32),
    )(acc, y, cnt)
    return out[:N_NODES]


def kernel(x, edge_index, W):
    B = x.shape[0]
    outs = []
    for i in range(B):
        ei = edge_index[i].astype(jnp.int32)
        outs.append(_gcn_single(x[i], ei[0], ei[1], W))
    return jnp.stack(outs)
